# baseline (speedup 1.0000x reference)
import functools

import jax
import jax.numpy as jnp
from jax import lax
from jax.experimental import pallas as pl
from jax.experimental.pallas import tpu as pltpu
from jax.experimental.pallas import tpu_sc as plsc

DIM = 300

_info = plsc.get_sparse_core_info()
NC, NS = _info.num_cores, _info.num_subcores
NW = NC * NS

CHUNK = 512


def _gather_kernel(n_rows):
    n_per_w = n_rows // NW
    n_chunks = n_per_w // CHUNK
    assert n_per_w % CHUNK == 0
    mesh = plsc.VectorSubcoreMesh(core_axis_name="c", subcore_axis_name="s")

    @functools.partial(
        pl.kernel,
        mesh=mesh,
        out_type=jax.ShapeDtypeStruct((n_rows, DIM), jnp.float32),
        scratch_types=[
            pltpu.VMEM((CHUNK,), jnp.int32),
            pltpu.SemaphoreType.DMA,
        ],
    )
    def k(table_hbm, idx_hbm, out_hbm, idx_v, sem):
        wid = lax.axis_index("s") * NC + lax.axis_index("c")
        base = wid * n_per_w

        def body(c, carry):
            off = base + c * CHUNK
            pltpu.sync_copy(idx_hbm.at[pl.ds(off, CHUNK)], idx_v)
            for t in range(CHUNK // 16):
                ivec = idx_v[pl.ds(t * 16, 16)]
                for l in range(16):
                    pltpu.async_copy(
                        table_hbm.at[ivec[l]],
                        out_hbm.at[off + t * 16 + l],
                        sem,
                    )
            pltpu.make_async_copy(
                table_hbm.at[pl.ds(0, CHUNK)],
                out_hbm.at[pl.ds(0, CHUNK)],
                sem,
            ).wait()
            return carry

        lax.fori_loop(0, n_chunks, body, 0)

    return k


def kernel(table, idxes):
    b, s = idxes.shape
    flat_idx = idxes.reshape(b * s)
    out = _gather_kernel(b * s)(table, flat_idx)
    return out.reshape(b, s, DIM)


# speedup vs baseline: 3.8451x; 3.8451x over previous
import functools

import jax
import jax.numpy as jnp
from jax import lax
from jax.experimental import pallas as pl
from jax.experimental.pallas import tpu as pltpu
from jax.experimental.pallas import tpu_sc as plsc

DIM = 300
DIMP = 304

_info = plsc.get_sparse_core_info()
NC, NS = _info.num_cores, _info.num_subcores
NW = NC * NS

CHUNK = 256


def _gather_kernel(n_rows):
    n_per_w = n_rows // NW
    n_chunks = n_per_w // CHUNK
    assert n_per_w % CHUNK == 0
    mesh = plsc.VectorSubcoreMesh(core_axis_name="c", subcore_axis_name="s")

    @functools.partial(
        pl.kernel,
        mesh=mesh,
        out_type=jax.ShapeDtypeStruct((n_rows, DIMP), jnp.float32),
        scratch_types=[
            pltpu.VMEM((CHUNK,), jnp.int32),
            pltpu.VMEM((CHUNK, DIMP), jnp.float32),
            pltpu.SemaphoreType.DMA,
        ],
        compiler_params=pltpu.CompilerParams(use_tc_tiling_on_sc=False),
    )
    def k(table_hbm, idx_hbm, out_hbm, idx_v, raw_v, sem):
        wid = lax.axis_index("s") * NC + lax.axis_index("c")
        base = wid * n_per_w

        def body(c, carry):
            off = base + c * CHUNK
            pltpu.sync_copy(idx_hbm.at[pl.ds(off, CHUNK)], idx_v)
            pltpu.async_copy(table_hbm.at[idx_v], raw_v, sem).wait()
            pltpu.sync_copy(raw_v, out_hbm.at[pl.ds(off, CHUNK)])
            return carry

        lax.fori_loop(0, n_chunks, body, 0)

    return k


def kernel(table, idxes):
    b, s = idxes.shape
    flat_idx = idxes.reshape(b * s)
    table_p = jnp.pad(table, ((0, 0), (0, DIMP - DIM)))
    out = _gather_kernel(b * s)(table_p, flat_idx)
    return out[:, :DIM].reshape(b, s, DIM)
